# Initial kernel scaffold; baseline (speedup 1.0000x reference)
import jax
import jax.numpy as jnp
from jax.experimental import pallas as pl


def kernel(x, edge_index, W1, a_src1, a_dst1, b1, W2, a_src2, a_dst2, b2):
    raise NotImplementedError("write your pallas kernel here")


# speedup vs baseline: 223.6847x; 223.6847x over previous
import functools

import jax
import jax.numpy as jnp
from jax import lax
from jax.experimental import pallas as pl
from jax.experimental.pallas import tpu as pltpu
from jax.experimental.pallas import tpu_sc as plsc

_SLOPE = 0.2


def _prep_body(x_ref, p_ref, o_ref):
    x0 = x_ref[:, 0:1]
    x1 = x_ref[:, 1:2]
    o_ref[...] = x0 * p_ref[0:1, :] + x1 * p_ref[1:2, :]


def _fin_body(a0_ref, a1_ref, nf_ref, x_ref, b_ref, xn_ref, ld_ref):
    nf = nf_ref[...]
    acc = a0_ref[...] + a1_ref[...]
    e1 = nf[:, 0:1] + nf[:, 1:2]
    e1 = jnp.where(e1 > 0, e1, _SLOPE * e1)
    ex1 = jnp.exp(e1)
    e2 = nf[:, 4:5] + nf[:, 5:6]
    e2 = jnp.where(e2 > 0, e2, _SLOPE * e2)
    ex2 = jnp.exp(e2)
    den1 = acc[:, 0:1] + ex1
    s0 = (acc[:, 1:2] + ex1 * nf[:, 2:3]) / den1 + b_ref[0:1, 0:1]
    s1 = (acc[:, 2:3] + ex1 * nf[:, 3:4]) / den1 + b_ref[0:1, 1:2]
    den2 = acc[:, 3:4] + ex2
    t0 = (acc[:, 4:5] + ex2 * nf[:, 6:7]) / den2 + b_ref[0:1, 2:3]
    t1 = (acc[:, 5:6] + ex2 * nf[:, 7:8]) / den2 + b_ref[0:1, 3:4]
    xn_ref[...] = jnp.concatenate(
        [x_ref[:, 2:3] * jnp.exp(s0) + t0, x_ref[:, 3:4] * jnp.exp(s1) + t1],
        axis=1)
    ld_ref[...] = s0 + s1


def _edge_body(R, B, KB, NBLK, nf_hbm, src_hbm, dst_hbm, out_hbm,
               src_idx, dst_idx, src_rows, dst_rows, out_rows, accum,
               sem_s, sem_d, sem_w):
    cid = lax.axis_index("c")
    sid = lax.axis_index("s")
    gwid = cid * 16 + sid

    zv = jnp.zeros((16,), jnp.float32)

    @pl.loop(0, B * 8 // 16)
    def _z(i):
        flat = i * 16 + lax.iota(jnp.int32, 16)
        plsc.store_scatter(out_rows, [flat >> 3, flat & 7], zv)

    ZR = R // 16
    z0 = pl.multiple_of(sid * ZR, 8)
    off = 0
    while off < ZR:
        sz = min(B, ZR - off)
        pltpu.sync_copy(out_rows.at[pl.ds(0, sz), :],
                        accum.at[pl.ds(z0 + off, sz), :])
        off += sz
    plsc.subcore_barrier()

    ept_rows = NBLK * KB
    base_row = gwid * ept_rows

    def col(c):
        return jnp.full((16,), c, jnp.int32)

    @pl.loop(0, NBLK)
    def _blk(blk):
        row0 = pl.multiple_of(base_row + blk * KB, 8)
        pltpu.sync_copy(src_hbm.at[pl.ds(row0, KB), :], src_idx)
        pltpu.sync_copy(dst_hbm.at[pl.ds(row0, KB), :], dst_idx)
        descs = []
        for j in range(KB):
            descs.append(pltpu.async_copy(
                nf_hbm.at[src_idx.at[j]],
                src_rows.at[pl.ds(j * 128, 128), :], sem_s))
            descs.append(pltpu.async_copy(
                nf_hbm.at[dst_idx.at[j]],
                dst_rows.at[pl.ds(j * 128, 128), :], sem_d))
        for d in descs:
            d.wait()

        @pl.loop(0, B // 16)
        def _grp(g):
            rows = g * 16 + lax.iota(jnp.int32, 16)
            as1 = plsc.load_gather(src_rows, [rows, col(0)])
            x10 = plsc.load_gather(src_rows, [rows, col(2)])
            x11 = plsc.load_gather(src_rows, [rows, col(3)])
            as2 = plsc.load_gather(src_rows, [rows, col(4)])
            x20 = plsc.load_gather(src_rows, [rows, col(6)])
            x21 = plsc.load_gather(src_rows, [rows, col(7)])
            ad1 = plsc.load_gather(dst_rows, [rows, col(1)])
            ad2 = plsc.load_gather(dst_rows, [rows, col(5)])
            e1 = as1 + ad1
            e1 = jnp.where(e1 > 0, e1, _SLOPE * e1)
            ex1 = jnp.exp(e1)
            e2 = as2 + ad2
            e2 = jnp.where(e2 > 0, e2, _SLOPE * e2)
            ex2 = jnp.exp(e2)
            plsc.store_scatter(out_rows, [rows, col(0)], ex1)
            plsc.store_scatter(out_rows, [rows, col(1)], ex1 * x10)
            plsc.store_scatter(out_rows, [rows, col(2)], ex1 * x11)
            plsc.store_scatter(out_rows, [rows, col(3)], ex2)
            plsc.store_scatter(out_rows, [rows, col(4)], ex2 * x20)
            plsc.store_scatter(out_rows, [rows, col(5)], ex2 * x21)

        wdescs = []
        for j in range(KB):
            wdescs.append(pltpu.async_copy(
                out_rows.at[pl.ds(j * 128, 128), :],
                accum.at[dst_idx.at[j]], sem_w, add=True))
        for d in wdescs:
            d.wait()

    plsc.subcore_barrier()

    w0 = pl.multiple_of(cid * R + z0, 8)
    off = 0
    while off < ZR:
        sz = min(B, ZR - off)
        pltpu.sync_copy(accum.at[pl.ds(z0 + off, sz), :],
                        out_hbm.at[pl.ds(w0 + off, sz), :])
        off += sz


def kernel(x, edge_index, W1, a_src1, a_dst1, b1, W2, a_src2, a_dst2, b2):
    N = x.shape[0]
    E = edge_index.shape[1]
    NW = 32
    B = 1024
    KB = B // 128
    EPT = -(-E // (NW * B)) * B
    Etot = EPT * NW
    NBLK = EPT // B
    R = -(-(N + 1) // 128) * 128

    P = jnp.zeros((2, 16), jnp.float32)
    P = P.at[:, 0].set(W1 @ a_src1).at[:, 1].set(W1 @ a_dst1)
    P = P.at[:, 2].set(W1[:, 0]).at[:, 3].set(W1[:, 1])
    P = P.at[:, 4].set(W2 @ a_src2).at[:, 5].set(W2 @ a_dst2)
    P = P.at[:, 6].set(W2[:, 0]).at[:, 7].set(W2[:, 1])
    bvec = jnp.concatenate([b1, b2]).reshape(1, 4).astype(jnp.float32)

    xp = jnp.pad(x, ((0, R - N), (0, 0)))
    src = edge_index[0]
    dst = edge_index[1]
    pad = Etot - E
    if pad:
        src = jnp.concatenate([src, jnp.zeros((pad,), edge_index.dtype)])
        dst = jnp.concatenate([dst, jnp.full((pad,), N, edge_index.dtype)])
    src_r = src.reshape(Etot // 128, 128)
    dst_r = dst.reshape(Etot // 128, 128)

    BR = 2048
    grid1 = -(-R // BR)
    nf = pl.pallas_call(
        _prep_body,
        grid=(grid1,),
        in_specs=[pl.BlockSpec((BR, 4), lambda i: (i, 0)),
                  pl.BlockSpec((2, 16), lambda i: (0, 0))],
        out_specs=pl.BlockSpec((BR, 16), lambda i: (i, 0)),
        out_shape=jax.ShapeDtypeStruct((R, 16), jnp.float32),
    )(xp, P)

    mesh = plsc.VectorSubcoreMesh(core_axis_name="c", subcore_axis_name="s")
    acc2 = pl.kernel(
        functools.partial(_edge_body, R, B, KB, NBLK),
        out_type=jax.ShapeDtypeStruct((2 * R, 8), jnp.float32),
        mesh=mesh,
        compiler_params=pltpu.CompilerParams(needs_layout_passes=False,
                                             use_tc_tiling_on_sc=False),
        scratch_types=[
            pltpu.VMEM((KB, 128), jnp.int32),
            pltpu.VMEM((KB, 128), jnp.int32),
            pltpu.VMEM((B, 16), jnp.float32),
            pltpu.VMEM((B, 16), jnp.float32),
            pltpu.VMEM((B, 8), jnp.float32),
            pltpu.VMEM_SHARED((R, 8), jnp.float32),
            pltpu.SemaphoreType.DMA,
            pltpu.SemaphoreType.DMA,
            pltpu.SemaphoreType.DMA,
        ],
    )(nf, src_r, dst_r)

    grid3 = -(-N // BR)
    x1_new, ld = pl.pallas_call(
        _fin_body,
        grid=(grid3,),
        in_specs=[pl.BlockSpec((BR, 8), lambda i: (i, 0)),
                  pl.BlockSpec((BR, 8), lambda i: (i, 0)),
                  pl.BlockSpec((BR, 16), lambda i: (i, 0)),
                  pl.BlockSpec((BR, 4), lambda i: (i, 0)),
                  pl.BlockSpec((1, 4), lambda i: (0, 0))],
        out_specs=[pl.BlockSpec((BR, 2), lambda i: (i, 0)),
                   pl.BlockSpec((BR, 1), lambda i: (i, 0))],
        out_shape=[jax.ShapeDtypeStruct((N, 2), jnp.float32),
                   jax.ShapeDtypeStruct((N, 1), jnp.float32)],
    )(acc2[:R], acc2[R:2 * R], nf, xp, bvec)

    return (x1_new, x[:, 2:], ld[:, 0])


# speedup vs baseline: 316.8682x; 1.4166x over previous
import functools

import jax
import jax.numpy as jnp
from jax import lax
from jax.experimental import pallas as pl
from jax.experimental.pallas import tpu as pltpu
from jax.experimental.pallas import tpu_sc as plsc

_SLOPE = 0.2


def _prep_body(x_ref, p_ref, o_ref):
    x0 = x_ref[:, 0:1]
    x1 = x_ref[:, 1:2]
    o_ref[...] = x0 * p_ref[0:1, :] + x1 * p_ref[1:2, :]


def _fin_body(a0_ref, a1_ref, nf_ref, x_ref, b_ref, xn_ref, ld_ref):
    nf = nf_ref[...]
    acc = a0_ref[...] + a1_ref[...]
    e1 = nf[:, 0:1] + nf[:, 1:2]
    e1 = jnp.where(e1 > 0, e1, _SLOPE * e1)
    ex1 = jnp.exp(e1)
    e2 = nf[:, 4:5] + nf[:, 5:6]
    e2 = jnp.where(e2 > 0, e2, _SLOPE * e2)
    ex2 = jnp.exp(e2)
    den1 = acc[:, 0:1] + ex1
    s0 = (acc[:, 1:2] + ex1 * nf[:, 2:3]) / den1 + b_ref[0:1, 0:1]
    s1 = (acc[:, 2:3] + ex1 * nf[:, 3:4]) / den1 + b_ref[0:1, 1:2]
    den2 = acc[:, 3:4] + ex2
    t0 = (acc[:, 4:5] + ex2 * nf[:, 6:7]) / den2 + b_ref[0:1, 2:3]
    t1 = (acc[:, 5:6] + ex2 * nf[:, 7:8]) / den2 + b_ref[0:1, 3:4]
    xn_ref[...] = jnp.concatenate(
        [x_ref[:, 2:3] * jnp.exp(s0) + t0, x_ref[:, 3:4] * jnp.exp(s1) + t1],
        axis=1)
    ld_ref[...] = s0 + s1


def _edge_body2(R, B, KB, NBLK, nf_hbm, src_hbm, dst_hbm, out_hbm,
                si0, si1, di0, di1, dc0, dc1, sr0, sr1, dr0, dr1,
                out_rows, accum,
                sem_i0, sem_i1, sem_c0, sem_c1, sem_g0, sem_g1,
                sem_h0, sem_h1, sem_w):
    si = (si0, si1)
    di = (di0, di1)
    dc = (dc0, dc1)
    sr = (sr0, sr1)
    dr = (dr0, dr1)
    sem_i = (sem_i0, sem_i1)
    sem_c = (sem_c0, sem_c1)
    sem_g = (sem_g0, sem_g1)
    sem_h = (sem_h0, sem_h1)

    cid = lax.axis_index("c")
    sid = lax.axis_index("s")
    gwid = cid * 16 + sid

    zv = jnp.zeros((16,), jnp.float32)

    @pl.loop(0, B * 8 // 16)
    def _z(i):
        flat = i * 16 + lax.iota(jnp.int32, 16)
        plsc.store_scatter(out_rows, [flat >> 3, flat & 7], zv)

    ZR = R // 16
    z0 = pl.multiple_of(sid * ZR, 8)
    off = 0
    while off < ZR:
        sz = min(B, ZR - off)
        pltpu.sync_copy(out_rows.at[pl.ds(0, sz), :],
                        accum.at[pl.ds(z0 + off, sz), :])
        off += sz
    plsc.subcore_barrier()

    ept_rows = NBLK * KB
    base_row = gwid * ept_rows

    def col(c):
        return jnp.full((16,), c, jnp.int32)

    def fire_idx(par, b):
        row0 = pl.multiple_of(base_row + b * KB, 8)
        pltpu.async_copy(src_hbm.at[pl.ds(row0, KB), :], si[par], sem_i[par])
        pltpu.async_copy(dst_hbm.at[pl.ds(row0, KB), :], di[par], sem_i[par])

    def wait_idx(par):
        pltpu.make_async_copy(src_hbm.at[pl.ds(0, KB), :], si[par],
                              sem_i[par]).wait()
        pltpu.make_async_copy(dst_hbm.at[pl.ds(0, KB), :], di[par],
                              sem_i[par]).wait()

    def fire_scidx(par, b):
        row0 = pl.multiple_of(base_row + b * KB, 8)
        pltpu.async_copy(dst_hbm.at[pl.ds(row0, KB), :], dc[par], sem_c[par])

    def wait_scidx(par):
        pltpu.make_async_copy(dst_hbm.at[pl.ds(0, KB), :], dc[par],
                              sem_c[par]).wait()

    def fire_gathers(par):
        for j in range(KB):
            pltpu.async_copy(nf_hbm.at[si[par].at[j]],
                             sr[par].at[pl.ds(j * 128, 128), :], sem_g[par])
            pltpu.async_copy(nf_hbm.at[di[par].at[j]],
                             dr[par].at[pl.ds(j * 128, 128), :], sem_h[par])

    def wait_gathers(par):
        for j in range(KB):
            pltpu.make_async_copy(nf_hbm.at[pl.ds(0, 128), :],
                                  sr[par].at[pl.ds(j * 128, 128), :],
                                  sem_g[par]).wait()
            pltpu.make_async_copy(nf_hbm.at[pl.ds(0, 128), :],
                                  dr[par].at[pl.ds(j * 128, 128), :],
                                  sem_h[par]).wait()

    def fire_scatter(par):
        for j in range(KB):
            pltpu.async_copy(out_rows.at[pl.ds(j * 128, 128), :],
                             accum.at[dc[par].at[j]], sem_w, add=True)

    def wait_scatter():
        for j in range(KB):
            pltpu.make_async_copy(out_rows.at[pl.ds(j * 128, 128), :],
                                  accum.at[pl.ds(0, 128), :], sem_w).wait()

    def compute(par):
        @pl.loop(0, B // 16)
        def _grp(g):
            rows = g * 16 + lax.iota(jnp.int32, 16)
            as1 = plsc.load_gather(sr[par], [rows, col(0)])
            x10 = plsc.load_gather(sr[par], [rows, col(2)])
            x11 = plsc.load_gather(sr[par], [rows, col(3)])
            as2 = plsc.load_gather(sr[par], [rows, col(4)])
            x20 = plsc.load_gather(sr[par], [rows, col(6)])
            x21 = plsc.load_gather(sr[par], [rows, col(7)])
            ad1 = plsc.load_gather(dr[par], [rows, col(1)])
            ad2 = plsc.load_gather(dr[par], [rows, col(5)])
            e1 = as1 + ad1
            e1 = jnp.where(e1 > 0, e1, _SLOPE * e1)
            ex1 = jnp.exp(e1)
            e2 = as2 + ad2
            e2 = jnp.where(e2 > 0, e2, _SLOPE * e2)
            ex2 = jnp.exp(e2)
            plsc.store_scatter(out_rows, [rows, col(0)], ex1)
            plsc.store_scatter(out_rows, [rows, col(1)], ex1 * x10)
            plsc.store_scatter(out_rows, [rows, col(2)], ex1 * x11)
            plsc.store_scatter(out_rows, [rows, col(3)], ex2)
            plsc.store_scatter(out_rows, [rows, col(4)], ex2 * x20)
            plsc.store_scatter(out_rows, [rows, col(5)], ex2 * x21)

    b0 = pl.multiple_of(base_row, 8)
    pltpu.sync_copy(src_hbm.at[pl.ds(b0, KB), :], si0)
    pltpu.sync_copy(dst_hbm.at[pl.ds(b0, KB), :], di0)
    pltpu.sync_copy(dst_hbm.at[pl.ds(b0, KB), :], dc0)
    fire_gathers(0)
    row1 = pl.multiple_of(base_row + KB, 8)
    pltpu.sync_copy(src_hbm.at[pl.ds(row1, KB), :], si1)
    pltpu.sync_copy(dst_hbm.at[pl.ds(row1, KB), :], di1)

    @pl.loop(0, NBLK // 2)
    def _pair(p):
        for phase in (0, 1):
            blk = 2 * p + phase
            par, npar = phase, 1 - phase

            if phase == 0:
                @pl.when(p > 0)
                def _():
                    wait_idx(npar)
            else:
                wait_idx(npar)
            fire_gathers(npar)
            wait_gathers(par)
            fire_idx(par, lax.rem(blk + 2, NBLK))
            if phase == 0:
                @pl.when(p > 0)
                def _():
                    wait_scatter()
            else:
                wait_scatter()
            fire_scidx(npar, lax.rem(blk + 1, NBLK))
            compute(par)
            if phase == 0:
                @pl.when(p > 0)
                def _():
                    wait_scidx(par)
            else:
                wait_scidx(par)
            fire_scatter(par)

    wait_scatter()
    wait_gathers(0)
    wait_idx(1)
    wait_scidx(0)
    plsc.subcore_barrier()

    w0 = pl.multiple_of(cid * R + z0, 8)
    off = 0
    while off < ZR:
        sz = min(B, ZR - off)
        pltpu.sync_copy(accum.at[pl.ds(z0 + off, sz), :],
                        out_hbm.at[pl.ds(w0 + off, sz), :])
        off += sz


def kernel(x, edge_index, W1, a_src1, a_dst1, b1, W2, a_src2, a_dst2, b2):
    N = x.shape[0]
    E = edge_index.shape[1]
    NW = 32
    B = 1024
    KB = B // 128
    EPT = -(-E // (NW * B)) * B
    Etot = EPT * NW
    NBLK = EPT // B
    R = -(-(N + 1) // 128) * 128

    P = jnp.zeros((2, 16), jnp.float32)
    P = P.at[:, 0].set(W1 @ a_src1).at[:, 1].set(W1 @ a_dst1)
    P = P.at[:, 2].set(W1[:, 0]).at[:, 3].set(W1[:, 1])
    P = P.at[:, 4].set(W2 @ a_src2).at[:, 5].set(W2 @ a_dst2)
    P = P.at[:, 6].set(W2[:, 0]).at[:, 7].set(W2[:, 1])
    bvec = jnp.concatenate([b1, b2]).reshape(1, 4).astype(jnp.float32)

    xp = jnp.pad(x, ((0, R - N), (0, 0)))
    src = edge_index[0]
    dst = edge_index[1]
    pad = Etot - E
    if pad:
        src = jnp.concatenate([src, jnp.zeros((pad,), edge_index.dtype)])
        dst = jnp.concatenate([dst, jnp.full((pad,), N, edge_index.dtype)])
    src_r = src.reshape(Etot // 128, 128)
    dst_r = dst.reshape(Etot // 128, 128)

    BR = 2048
    grid1 = -(-R // BR)
    nf = pl.pallas_call(
        _prep_body,
        grid=(grid1,),
        in_specs=[pl.BlockSpec((BR, 4), lambda i: (i, 0)),
                  pl.BlockSpec((2, 16), lambda i: (0, 0))],
        out_specs=pl.BlockSpec((BR, 16), lambda i: (i, 0)),
        out_shape=jax.ShapeDtypeStruct((R, 16), jnp.float32),
    )(xp, P)

    mesh = plsc.VectorSubcoreMesh(core_axis_name="c", subcore_axis_name="s")
    acc2 = pl.kernel(
        functools.partial(_edge_body2, R, B, KB, NBLK),
        out_type=jax.ShapeDtypeStruct((2 * R, 8), jnp.float32),
        mesh=mesh,
        compiler_params=pltpu.CompilerParams(needs_layout_passes=False,
                                             use_tc_tiling_on_sc=False),
        scratch_types=[
            pltpu.VMEM((KB, 128), jnp.int32),
            pltpu.VMEM((KB, 128), jnp.int32),
            pltpu.VMEM((KB, 128), jnp.int32),
            pltpu.VMEM((KB, 128), jnp.int32),
            pltpu.VMEM((KB, 128), jnp.int32),
            pltpu.VMEM((KB, 128), jnp.int32),
            pltpu.VMEM((B, 16), jnp.float32),
            pltpu.VMEM((B, 16), jnp.float32),
            pltpu.VMEM((B, 16), jnp.float32),
            pltpu.VMEM((B, 16), jnp.float32),
            pltpu.VMEM((B, 8), jnp.float32),
            pltpu.VMEM_SHARED((R, 8), jnp.float32),
        ] + [pltpu.SemaphoreType.DMA] * 9,
    )(nf, src_r, dst_r)

    grid3 = -(-N // BR)
    x1_new, ld = pl.pallas_call(
        _fin_body,
        grid=(grid3,),
        in_specs=[pl.BlockSpec((BR, 8), lambda i: (i, 0)),
                  pl.BlockSpec((BR, 8), lambda i: (i, 0)),
                  pl.BlockSpec((BR, 16), lambda i: (i, 0)),
                  pl.BlockSpec((BR, 4), lambda i: (i, 0)),
                  pl.BlockSpec((1, 4), lambda i: (0, 0))],
        out_specs=[pl.BlockSpec((BR, 2), lambda i: (i, 0)),
                   pl.BlockSpec((BR, 1), lambda i: (i, 0))],
        out_shape=[jax.ShapeDtypeStruct((N, 2), jnp.float32),
                   jax.ShapeDtypeStruct((N, 1), jnp.float32)],
    )(acc2[:R], acc2[R:2 * R], nf, xp, bvec)

    return (x1_new, x[:, 2:], ld[:, 0])


# speedup vs baseline: 434.9806x; 1.3727x over previous
import functools

import jax
import jax.numpy as jnp
from jax import lax
from jax.experimental import pallas as pl
from jax.experimental.pallas import tpu as pltpu
from jax.experimental.pallas import tpu_sc as plsc

_SLOPE = 0.2


def _prep_body(x_ref, xt_ref, p_ref, pt_ref, nf_ref, nft_ref, x2t_ref):
    x0 = x_ref[:, 0:1]
    x1 = x_ref[:, 1:2]
    nf_ref[...] = x0 * p_ref[0:1, :] + x1 * p_ref[1:2, :]
    x0r = xt_ref[0:1, :]
    x1r = xt_ref[1:2, :]
    nft_ref[...] = pt_ref[:, 0:1] * x0r + pt_ref[:, 1:2] * x1r
    x2t_ref[...] = xt_ref[2:4, :]


def _fin_body(acc_ref, nft_ref, x2t_ref, b_ref, xnt_ref, ld_ref):
    def acc(c):
        return acc_ref[c:c + 1, :] + acc_ref[c + 8:c + 9, :]

    as1 = nft_ref[0:1, :]
    ad1 = nft_ref[1:2, :]
    x10 = nft_ref[2:3, :]
    x11 = nft_ref[3:4, :]
    as2 = nft_ref[4:5, :]
    ad2 = nft_ref[5:6, :]
    x20 = nft_ref[6:7, :]
    x21 = nft_ref[7:8, :]
    e1 = as1 + ad1
    e1 = jnp.where(e1 > 0, e1, _SLOPE * e1)
    ex1 = jnp.exp(e1)
    e2 = as2 + ad2
    e2 = jnp.where(e2 > 0, e2, _SLOPE * e2)
    ex2 = jnp.exp(e2)
    den1 = acc(0) + ex1
    s0 = (acc(1) + ex1 * x10) / den1 + b_ref[0:1, 0:1]
    s1 = (acc(2) + ex1 * x11) / den1 + b_ref[0:1, 1:2]
    den2 = acc(3) + ex2
    t0 = (acc(4) + ex2 * x20) / den2 + b_ref[0:1, 2:3]
    t1 = (acc(5) + ex2 * x21) / den2 + b_ref[0:1, 3:4]
    xnt_ref[...] = jnp.concatenate(
        [x2t_ref[0:1, :] * jnp.exp(s0) + t0,
         x2t_ref[1:2, :] * jnp.exp(s1) + t1], axis=0)
    ld_ref[...] = (s0 + s1)[0, :]


def _edge_body2(R, B, KB, NBLK, nf_hbm, src_hbm, dst_hbm, out_hbm,
                si0, si1, di0, di1, dc0, dc1, sr0, sr1, dr0, dr1,
                out_rows, accum,
                sem_i0, sem_i1, sem_c0, sem_c1, sem_g0, sem_g1,
                sem_h0, sem_h1, sem_w):
    si = (si0, si1)
    di = (di0, di1)
    dc = (dc0, dc1)
    sr = (sr0, sr1)
    dr = (dr0, dr1)
    sem_i = (sem_i0, sem_i1)
    sem_c = (sem_c0, sem_c1)
    sem_g = (sem_g0, sem_g1)
    sem_h = (sem_h0, sem_h1)

    cid = lax.axis_index("c")
    sid = lax.axis_index("s")
    gwid = cid * 16 + sid

    zv = jnp.zeros((16,), jnp.float32)

    @pl.loop(0, B * 8 // 16)
    def _z(i):
        flat = i * 16 + lax.iota(jnp.int32, 16)
        plsc.store_scatter(out_rows, [flat >> 3, flat & 7], zv)

    ZR = R // 16
    z0 = pl.multiple_of(sid * ZR, 8)
    off = 0
    while off < ZR:
        sz = min(B, ZR - off)
        pltpu.sync_copy(out_rows.at[pl.ds(0, sz), :],
                        accum.at[pl.ds(z0 + off, sz), :])
        off += sz
    plsc.subcore_barrier()

    ept_rows = NBLK * KB
    base_row = gwid * ept_rows

    def col(c):
        return jnp.full((16,), c, jnp.int32)

    def fire_idx(par, b):
        row0 = pl.multiple_of(base_row + b * KB, 8)
        pltpu.async_copy(src_hbm.at[pl.ds(row0, KB), :], si[par], sem_i[par])
        pltpu.async_copy(dst_hbm.at[pl.ds(row0, KB), :], di[par], sem_i[par])

    def wait_idx(par):
        pltpu.make_async_copy(src_hbm.at[pl.ds(0, KB), :], si[par],
                              sem_i[par]).wait()
        pltpu.make_async_copy(dst_hbm.at[pl.ds(0, KB), :], di[par],
                              sem_i[par]).wait()

    def fire_scidx(par, b):
        row0 = pl.multiple_of(base_row + b * KB, 8)
        pltpu.async_copy(dst_hbm.at[pl.ds(row0, KB), :], dc[par], sem_c[par])

    def wait_scidx(par):
        pltpu.make_async_copy(dst_hbm.at[pl.ds(0, KB), :], dc[par],
                              sem_c[par]).wait()

    def fire_gathers(par):
        for j in range(KB):
            pltpu.async_copy(nf_hbm.at[si[par].at[j]],
                             sr[par].at[pl.ds(j * 128, 128), :], sem_g[par])
            pltpu.async_copy(nf_hbm.at[di[par].at[j]],
                             dr[par].at[pl.ds(j * 128, 128), :], sem_h[par])

    def wait_gathers(par):
        for j in range(KB):
            pltpu.make_async_copy(nf_hbm.at[pl.ds(0, 128), :],
                                  sr[par].at[pl.ds(j * 128, 128), :],
                                  sem_g[par]).wait()
            pltpu.make_async_copy(nf_hbm.at[pl.ds(0, 128), :],
                                  dr[par].at[pl.ds(j * 128, 128), :],
                                  sem_h[par]).wait()

    def fire_scatter(par):
        for j in range(KB):
            pltpu.async_copy(out_rows.at[pl.ds(j * 128, 128), :],
                             accum.at[dc[par].at[j]], sem_w, add=True)

    def wait_scatter():
        for j in range(KB):
            pltpu.make_async_copy(out_rows.at[pl.ds(j * 128, 128), :],
                                  accum.at[pl.ds(0, 128), :], sem_w).wait()

    def compute(par):
        @pl.loop(0, B // 16)
        def _grp(g):
            rows = g * 16 + lax.iota(jnp.int32, 16)
            as1 = plsc.load_gather(sr[par], [rows, col(0)])
            x10 = plsc.load_gather(sr[par], [rows, col(2)])
            x11 = plsc.load_gather(sr[par], [rows, col(3)])
            as2 = plsc.load_gather(sr[par], [rows, col(4)])
            x20 = plsc.load_gather(sr[par], [rows, col(6)])
            x21 = plsc.load_gather(sr[par], [rows, col(7)])
            ad1 = plsc.load_gather(dr[par], [rows, col(1)])
            ad2 = plsc.load_gather(dr[par], [rows, col(5)])
            e1 = as1 + ad1
            e1 = jnp.where(e1 > 0, e1, _SLOPE * e1)
            ex1 = jnp.exp(e1)
            e2 = as2 + ad2
            e2 = jnp.where(e2 > 0, e2, _SLOPE * e2)
            ex2 = jnp.exp(e2)
            plsc.store_scatter(out_rows, [rows, col(0)], ex1)
            plsc.store_scatter(out_rows, [rows, col(1)], ex1 * x10)
            plsc.store_scatter(out_rows, [rows, col(2)], ex1 * x11)
            plsc.store_scatter(out_rows, [rows, col(3)], ex2)
            plsc.store_scatter(out_rows, [rows, col(4)], ex2 * x20)
            plsc.store_scatter(out_rows, [rows, col(5)], ex2 * x21)

    b0 = pl.multiple_of(base_row, 8)
    pltpu.sync_copy(src_hbm.at[pl.ds(b0, KB), :], si0)
    pltpu.sync_copy(dst_hbm.at[pl.ds(b0, KB), :], di0)
    pltpu.sync_copy(dst_hbm.at[pl.ds(b0, KB), :], dc0)
    fire_gathers(0)
    row1 = pl.multiple_of(base_row + KB, 8)
    pltpu.sync_copy(src_hbm.at[pl.ds(row1, KB), :], si1)
    pltpu.sync_copy(dst_hbm.at[pl.ds(row1, KB), :], di1)

    @pl.loop(0, NBLK // 2)
    def _pair(p):
        for phase in (0, 1):
            blk = 2 * p + phase
            par, npar = phase, 1 - phase

            if phase == 0:
                @pl.when(p > 0)
                def _():
                    wait_idx(npar)
            else:
                wait_idx(npar)
            fire_gathers(npar)
            wait_gathers(par)
            fire_idx(par, lax.rem(blk + 2, NBLK))
            if phase == 0:
                @pl.when(p > 0)
                def _():
                    wait_scatter()
            else:
                wait_scatter()
            fire_scidx(npar, lax.rem(blk + 1, NBLK))
            compute(par)
            if phase == 0:
                @pl.when(p > 0)
                def _():
                    wait_scidx(par)
            else:
                wait_scidx(par)
            fire_scatter(par)

    wait_scatter()
    wait_gathers(0)
    wait_idx(1)
    wait_scidx(0)
    plsc.subcore_barrier()

    w0 = pl.multiple_of(cid * R + z0, 8)
    off = 0
    while off < ZR:
        sz = min(B, ZR - off)
        pltpu.sync_copy(accum.at[pl.ds(z0 + off, sz), :],
                        out_hbm.at[pl.ds(w0 + off, sz), :])
        off += sz


def kernel(x, edge_index, W1, a_src1, a_dst1, b1, W2, a_src2, a_dst2, b2):
    N = x.shape[0]
    E = edge_index.shape[1]
    NW = 32
    B = 1024
    KB = B // 128
    EPT = -(-E // (NW * B)) * B
    Etot = EPT * NW
    NBLK = EPT // B
    R = -(-(N + 1) // 128) * 128

    P = jnp.zeros((2, 16), jnp.float32)
    P = P.at[:, 0].set(W1 @ a_src1).at[:, 1].set(W1 @ a_dst1)
    P = P.at[:, 2].set(W1[:, 0]).at[:, 3].set(W1[:, 1])
    P = P.at[:, 4].set(W2 @ a_src2).at[:, 5].set(W2 @ a_dst2)
    P = P.at[:, 6].set(W2[:, 0]).at[:, 7].set(W2[:, 1])
    PT = P[:, :8].T
    bvec = jnp.concatenate([b1, b2]).reshape(1, 4).astype(jnp.float32)

    xp = jnp.pad(x, ((0, R - N), (0, 0)))
    xpT = xp.T
    src = edge_index[0]
    dst = edge_index[1]
    pad = Etot - E
    if pad:
        src = jnp.concatenate([src, jnp.zeros((pad,), edge_index.dtype)])
        dst = jnp.concatenate([dst, jnp.full((pad,), N, edge_index.dtype)])
    src_r = src.reshape(Etot // 128, 128)
    dst_r = dst.reshape(Etot // 128, 128)

    BR = 4096
    grid1 = -(-R // BR)
    nf, nfT, x2T = pl.pallas_call(
        _prep_body,
        grid=(grid1,),
        in_specs=[pl.BlockSpec((BR, 4), lambda i: (i, 0)),
                  pl.BlockSpec((4, BR), lambda i: (0, i)),
                  pl.BlockSpec((2, 16), lambda i: (0, 0)),
                  pl.BlockSpec((8, 2), lambda i: (0, 0))],
        out_specs=[pl.BlockSpec((BR, 16), lambda i: (i, 0)),
                   pl.BlockSpec((8, BR), lambda i: (0, i)),
                   pl.BlockSpec((2, BR), lambda i: (0, i))],
        out_shape=[jax.ShapeDtypeStruct((R, 16), jnp.float32),
                   jax.ShapeDtypeStruct((8, R), jnp.float32),
                   jax.ShapeDtypeStruct((2, R), jnp.float32)],
    )(xp, xpT, P, PT)

    mesh = plsc.VectorSubcoreMesh(core_axis_name="c", subcore_axis_name="s")
    acc2 = pl.kernel(
        functools.partial(_edge_body2, R, B, KB, NBLK),
        out_type=jax.ShapeDtypeStruct((2 * R, 8), jnp.float32),
        mesh=mesh,
        compiler_params=pltpu.CompilerParams(needs_layout_passes=False,
                                             use_tc_tiling_on_sc=False),
        scratch_types=[
            pltpu.VMEM((KB, 128), jnp.int32),
            pltpu.VMEM((KB, 128), jnp.int32),
            pltpu.VMEM((KB, 128), jnp.int32),
            pltpu.VMEM((KB, 128), jnp.int32),
            pltpu.VMEM((KB, 128), jnp.int32),
            pltpu.VMEM((KB, 128), jnp.int32),
            pltpu.VMEM((B, 16), jnp.float32),
            pltpu.VMEM((B, 16), jnp.float32),
            pltpu.VMEM((B, 16), jnp.float32),
            pltpu.VMEM((B, 16), jnp.float32),
            pltpu.VMEM((B, 8), jnp.float32),
            pltpu.VMEM_SHARED((R, 8), jnp.float32),
        ] + [pltpu.SemaphoreType.DMA] * 9,
    )(nf, src_r, dst_r)

    accT = jnp.transpose(acc2.reshape(2, R, 8), (0, 2, 1)).reshape(16, R)
    grid3 = -(-N // BR)
    x1nT, ld = pl.pallas_call(
        _fin_body,
        grid=(grid3,),
        in_specs=[pl.BlockSpec((16, BR), lambda i: (0, i)),
                  pl.BlockSpec((8, BR), lambda i: (0, i)),
                  pl.BlockSpec((2, BR), lambda i: (0, i)),
                  pl.BlockSpec((1, 4), lambda i: (0, 0))],
        out_specs=[pl.BlockSpec((2, BR), lambda i: (0, i)),
                   pl.BlockSpec((BR,), lambda i: (i,))],
        out_shape=[jax.ShapeDtypeStruct((2, N), jnp.float32),
                   jax.ShapeDtypeStruct((N,), jnp.float32)],
    )(accT, nfT, x2T, bvec)

    return (x1nT.T, x[:, 2:], ld)


# speedup vs baseline: 455.2065x; 1.0465x over previous
import functools

import jax
import jax.numpy as jnp
from jax import lax
from jax.experimental import pallas as pl
from jax.experimental.pallas import tpu as pltpu
from jax.experimental.pallas import tpu_sc as plsc

_SLOPE = 0.2


def _prep_body(x_ref, xt_ref, p_ref, pt_ref, nf_ref, nft_ref, x2t_ref):
    x0 = x_ref[:, 0:1]
    x1 = x_ref[:, 1:2]
    nf_ref[...] = x0 * p_ref[0:1, :] + x1 * p_ref[1:2, :]
    x0r = xt_ref[0:1, :]
    x1r = xt_ref[1:2, :]
    nft_ref[...] = pt_ref[:, 0:1] * x0r + pt_ref[:, 1:2] * x1r
    x2t_ref[...] = xt_ref[2:4, :]


def _fin_body(acc_ref, nft_ref, x2t_ref, b_ref, xnt_ref, ld_ref):
    def acc(c):
        return acc_ref[c:c + 1, :] + acc_ref[c + 8:c + 9, :]

    as1 = nft_ref[0:1, :]
    ad1 = nft_ref[1:2, :]
    x10 = nft_ref[2:3, :]
    x11 = nft_ref[3:4, :]
    as2 = nft_ref[4:5, :]
    ad2 = nft_ref[5:6, :]
    x20 = nft_ref[6:7, :]
    x21 = nft_ref[7:8, :]
    e1 = as1 + ad1
    e1 = jnp.where(e1 > 0, e1, _SLOPE * e1)
    ex1 = jnp.exp(e1)
    e2 = as2 + ad2
    e2 = jnp.where(e2 > 0, e2, _SLOPE * e2)
    ex2 = jnp.exp(e2)
    den1 = acc(0) + ex1
    s0 = (acc(1) + ex1 * x10) / den1 + b_ref[0:1, 0:1]
    s1 = (acc(2) + ex1 * x11) / den1 + b_ref[0:1, 1:2]
    den2 = acc(3) + ex2
    t0 = (acc(4) + ex2 * x20) / den2 + b_ref[0:1, 2:3]
    t1 = (acc(5) + ex2 * x21) / den2 + b_ref[0:1, 3:4]
    xnt_ref[...] = jnp.concatenate(
        [x2t_ref[0:1, :] * jnp.exp(s0) + t0,
         x2t_ref[1:2, :] * jnp.exp(s1) + t1], axis=0)
    ld_ref[...] = (s0 + s1)[0, :]


def _edge_body2(R, B, KB, NB0, NB1, nf_hbm, src_hbm, dst_hbm, out_hbm,
                si0, si1, di0, di1, dc0, dc1, sr0, sr1, dr0, dr1,
                out_rows, accum,
                sem_i0, sem_i1, sem_c0, sem_c1, sem_g0, sem_g1,
                sem_h0, sem_h1, sem_w):
    si = (si0, si1)
    di = (di0, di1)
    dc = (dc0, dc1)
    sr = (sr0, sr1)
    dr = (dr0, dr1)
    sem_i = (sem_i0, sem_i1)
    sem_c = (sem_c0, sem_c1)
    sem_g = (sem_g0, sem_g1)
    sem_h = (sem_h0, sem_h1)

    cid = lax.axis_index("c")
    sid = lax.axis_index("s")
    gwid = cid * 16 + sid

    zv = jnp.zeros((16,), jnp.float32)

    @pl.loop(0, B * 8 // 16)
    def _z(i):
        flat = i * 16 + lax.iota(jnp.int32, 16)
        plsc.store_scatter(out_rows, [flat >> 3, flat & 7], zv)

    ZR = R // 16
    z0 = pl.multiple_of(sid * ZR, 8)
    off = 0
    while off < ZR:
        sz = min(B, ZR - off)
        pltpu.sync_copy(out_rows.at[pl.ds(0, sz), :],
                        accum.at[pl.ds(z0 + off, sz), :])
        off += sz
    plsc.subcore_barrier()

    nblk = NB0 - cid * (NB0 - NB1)
    base_blk = cid * 16 * NB0 + sid * nblk
    base_row = pl.multiple_of(base_blk * KB, 8)

    def col(c):
        return jnp.full((16,), c, jnp.int32)

    def fire_idx(par, b):
        row0 = pl.multiple_of(base_row + b * KB, 8)
        pltpu.async_copy(src_hbm.at[pl.ds(row0, KB), :], si[par], sem_i[par])
        pltpu.async_copy(dst_hbm.at[pl.ds(row0, KB), :], di[par], sem_i[par])

    def wait_idx(par):
        pltpu.make_async_copy(src_hbm.at[pl.ds(0, KB), :], si[par],
                              sem_i[par]).wait()
        pltpu.make_async_copy(dst_hbm.at[pl.ds(0, KB), :], di[par],
                              sem_i[par]).wait()

    def fire_scidx(par, b):
        row0 = pl.multiple_of(base_row + b * KB, 8)
        pltpu.async_copy(dst_hbm.at[pl.ds(row0, KB), :], dc[par], sem_c[par])

    def wait_scidx(par):
        pltpu.make_async_copy(dst_hbm.at[pl.ds(0, KB), :], dc[par],
                              sem_c[par]).wait()

    def fire_gathers(par):
        for j in range(KB):
            pltpu.async_copy(nf_hbm.at[si[par].at[j]],
                             sr[par].at[pl.ds(j * 128, 128), :], sem_g[par])
            pltpu.async_copy(nf_hbm.at[di[par].at[j]],
                             dr[par].at[pl.ds(j * 128, 128), :], sem_h[par])

    def wait_gathers(par):
        for j in range(KB):
            pltpu.make_async_copy(nf_hbm.at[pl.ds(0, 128), :],
                                  sr[par].at[pl.ds(j * 128, 128), :],
                                  sem_g[par]).wait()
            pltpu.make_async_copy(nf_hbm.at[pl.ds(0, 128), :],
                                  dr[par].at[pl.ds(j * 128, 128), :],
                                  sem_h[par]).wait()

    def fire_scatter(par):
        for j in range(KB):
            pltpu.async_copy(out_rows.at[pl.ds(j * 128, 128), :],
                             accum.at[dc[par].at[j]], sem_w, add=True)

    def wait_scatter():
        for j in range(KB):
            pltpu.make_async_copy(out_rows.at[pl.ds(j * 128, 128), :],
                                  accum.at[pl.ds(0, 128), :], sem_w).wait()

    def compute(par):
        @pl.loop(0, B // 16)
        def _grp(g):
            rows = g * 16 + lax.iota(jnp.int32, 16)
            as1 = plsc.load_gather(sr[par], [rows, col(0)])
            x10 = plsc.load_gather(sr[par], [rows, col(2)])
            x11 = plsc.load_gather(sr[par], [rows, col(3)])
            as2 = plsc.load_gather(sr[par], [rows, col(4)])
            x20 = plsc.load_gather(sr[par], [rows, col(6)])
            x21 = plsc.load_gather(sr[par], [rows, col(7)])
            ad1 = plsc.load_gather(dr[par], [rows, col(1)])
            ad2 = plsc.load_gather(dr[par], [rows, col(5)])
            e1 = as1 + ad1
            e1 = jnp.where(e1 > 0, e1, _SLOPE * e1)
            ex1 = jnp.exp(e1)
            e2 = as2 + ad2
            e2 = jnp.where(e2 > 0, e2, _SLOPE * e2)
            ex2 = jnp.exp(e2)
            plsc.store_scatter(out_rows, [rows, col(0)], ex1)
            plsc.store_scatter(out_rows, [rows, col(1)], ex1 * x10)
            plsc.store_scatter(out_rows, [rows, col(2)], ex1 * x11)
            plsc.store_scatter(out_rows, [rows, col(3)], ex2)
            plsc.store_scatter(out_rows, [rows, col(4)], ex2 * x20)
            plsc.store_scatter(out_rows, [rows, col(5)], ex2 * x21)

    b0 = pl.multiple_of(base_row, 8)
    pltpu.sync_copy(src_hbm.at[pl.ds(b0, KB), :], si0)
    pltpu.sync_copy(dst_hbm.at[pl.ds(b0, KB), :], di0)
    pltpu.sync_copy(dst_hbm.at[pl.ds(b0, KB), :], dc0)
    fire_gathers(0)
    row1 = pl.multiple_of(base_row + KB, 8)
    pltpu.sync_copy(src_hbm.at[pl.ds(row1, KB), :], si1)
    pltpu.sync_copy(dst_hbm.at[pl.ds(row1, KB), :], di1)

    def pair_body(p):
        for phase in (0, 1):
            blk = 2 * p + phase
            par, npar = phase, 1 - phase

            if phase == 0:
                @pl.when(p > 0)
                def _():
                    wait_idx(npar)
            else:
                wait_idx(npar)
            fire_gathers(npar)
            wait_gathers(par)
            fire_idx(par, jnp.minimum(blk + 2, nblk - 1))
            if phase == 0:
                @pl.when(p > 0)
                def _():
                    wait_scatter()
            else:
                wait_scatter()
            fire_scidx(npar, jnp.minimum(blk + 1, nblk - 1))
            compute(par)
            if phase == 0:
                @pl.when(p > 0)
                def _():
                    wait_scidx(par)
            else:
                wait_scidx(par)
            fire_scatter(par)

    @pl.loop(0, NB1 // 2)
    def _pair_common(p):
        pair_body(p)

    @pl.when(cid == 0)
    def _extra_pairs():
        @pl.loop(0, (NB0 - NB1) // 2)
        def _pair_extra(q):
            pair_body(q + NB1 // 2)

    wait_scatter()
    wait_gathers(0)
    wait_idx(1)
    wait_scidx(0)
    plsc.subcore_barrier()

    w0 = pl.multiple_of(cid * R + z0, 8)
    off = 0
    while off < ZR:
        sz = min(B, ZR - off)
        pltpu.sync_copy(accum.at[pl.ds(z0 + off, sz), :],
                        out_hbm.at[pl.ds(w0 + off, sz), :])
        off += sz


def kernel(x, edge_index, W1, a_src1, a_dst1, b1, W2, a_src2, a_dst2, b2):
    N = x.shape[0]
    E = edge_index.shape[1]
    NW = 32
    B = 1024
    KB = B // 128
    EPT = -(-E // (NW * B)) * B
    Etot = EPT * NW
    per_pair = Etot // B // 16
    NB0 = 2 * round(per_pair * 4 / 7 / 2)
    NB1 = per_pair - NB0
    if NB1 % 2:
        NB0 -= 1
        NB1 += 1
    assert NB0 >= NB1 > 0 and NB0 % 2 == 0 and NB1 % 2 == 0
    R = -(-(N + 1) // 128) * 128

    P = jnp.zeros((2, 16), jnp.float32)
    P = P.at[:, 0].set(W1 @ a_src1).at[:, 1].set(W1 @ a_dst1)
    P = P.at[:, 2].set(W1[:, 0]).at[:, 3].set(W1[:, 1])
    P = P.at[:, 4].set(W2 @ a_src2).at[:, 5].set(W2 @ a_dst2)
    P = P.at[:, 6].set(W2[:, 0]).at[:, 7].set(W2[:, 1])
    PT = P[:, :8].T
    bvec = jnp.concatenate([b1, b2]).reshape(1, 4).astype(jnp.float32)

    xp = jnp.pad(x, ((0, R - N), (0, 0)))
    xpT = xp.T
    src = edge_index[0]
    dst = edge_index[1]
    pad = Etot - E
    if pad:
        src = jnp.concatenate([src, jnp.zeros((pad,), edge_index.dtype)])
        dst = jnp.concatenate([dst, jnp.full((pad,), N, edge_index.dtype)])
    src_r = src.reshape(Etot // 128, 128)
    dst_r = dst.reshape(Etot // 128, 128)

    BR = 4096
    grid1 = -(-R // BR)
    nf, nfT, x2T = pl.pallas_call(
        _prep_body,
        grid=(grid1,),
        in_specs=[pl.BlockSpec((BR, 4), lambda i: (i, 0)),
                  pl.BlockSpec((4, BR), lambda i: (0, i)),
                  pl.BlockSpec((2, 16), lambda i: (0, 0)),
                  pl.BlockSpec((8, 2), lambda i: (0, 0))],
        out_specs=[pl.BlockSpec((BR, 16), lambda i: (i, 0)),
                   pl.BlockSpec((8, BR), lambda i: (0, i)),
                   pl.BlockSpec((2, BR), lambda i: (0, i))],
        out_shape=[jax.ShapeDtypeStruct((R, 16), jnp.float32),
                   jax.ShapeDtypeStruct((8, R), jnp.float32),
                   jax.ShapeDtypeStruct((2, R), jnp.float32)],
    )(xp, xpT, P, PT)

    mesh = plsc.VectorSubcoreMesh(core_axis_name="c", subcore_axis_name="s")
    acc2 = pl.kernel(
        functools.partial(_edge_body2, R, B, KB, NB0, NB1),
        out_type=jax.ShapeDtypeStruct((2 * R, 8), jnp.float32),
        mesh=mesh,
        compiler_params=pltpu.CompilerParams(needs_layout_passes=False,
                                             use_tc_tiling_on_sc=False),
        scratch_types=[
            pltpu.VMEM((KB, 128), jnp.int32),
            pltpu.VMEM((KB, 128), jnp.int32),
            pltpu.VMEM((KB, 128), jnp.int32),
            pltpu.VMEM((KB, 128), jnp.int32),
            pltpu.VMEM((KB, 128), jnp.int32),
            pltpu.VMEM((KB, 128), jnp.int32),
            pltpu.VMEM((B, 16), jnp.float32),
            pltpu.VMEM((B, 16), jnp.float32),
            pltpu.VMEM((B, 16), jnp.float32),
            pltpu.VMEM((B, 16), jnp.float32),
            pltpu.VMEM((B, 8), jnp.float32),
            pltpu.VMEM_SHARED((R, 8), jnp.float32),
        ] + [pltpu.SemaphoreType.DMA] * 9,
    )(nf, src_r, dst_r)

    accT = jnp.transpose(acc2.reshape(2, R, 8), (0, 2, 1)).reshape(16, R)
    grid3 = -(-N // BR)
    x1nT, ld = pl.pallas_call(
        _fin_body,
        grid=(grid3,),
        in_specs=[pl.BlockSpec((16, BR), lambda i: (0, i)),
                  pl.BlockSpec((8, BR), lambda i: (0, i)),
                  pl.BlockSpec((2, BR), lambda i: (0, i)),
                  pl.BlockSpec((1, 4), lambda i: (0, 0))],
        out_specs=[pl.BlockSpec((2, BR), lambda i: (0, i)),
                   pl.BlockSpec((BR,), lambda i: (i,))],
        out_shape=[jax.ShapeDtypeStruct((2, N), jnp.float32),
                   jax.ShapeDtypeStruct((N,), jnp.float32)],
    )(accT, nfT, x2T, bvec)

    return (x1nT.T, x[:, 2:], ld)
